# Initial kernel scaffold; baseline (speedup 1.0000x reference)
import jax
import jax.numpy as jnp
from jax.experimental import pallas as pl


def kernel(input_ids, table):
    raise NotImplementedError("write your pallas kernel here")


# speedup vs baseline: 2.4905x; 2.4905x over previous
import functools

import jax
import jax.numpy as jnp
from jax import lax
from jax.experimental import pallas as pl
from jax.experimental.pallas import tpu as pltpu
from jax.experimental.pallas import tpu_sc as plsc

_INFO = plsc.get_sparse_core_info()
_NC = _INFO.num_cores
_NS = _INFO.num_subcores
_NW = _NC * _NS

_CHUNK = 2048


@functools.partial(jax.jit, static_argnums=(2, 3))
def _gather(flat_idx, table, b_per_w, num_chunks):
    D = table.shape[1]
    B = flat_idx.shape[0]
    mesh = plsc.VectorSubcoreMesh(core_axis_name="c", subcore_axis_name="s")

    @functools.partial(
        pl.kernel,
        mesh=mesh,
        out_type=jax.ShapeDtypeStruct((B, D), jnp.float32),
        scratch_types=[
            pltpu.VMEM((_CHUNK,), jnp.int32),
            pltpu.VMEM((_CHUNK, D), jnp.float32),
            pltpu.SemaphoreType.DMA,
        ],
        compiler_params=pltpu.CompilerParams(use_tc_tiling_on_sc=False),
    )
    def k(idx_hbm, table_hbm, out_hbm, idx_v, rows_v, sem):
        wid = lax.axis_index("s") * _NC + lax.axis_index("c")
        base = wid * b_per_w

        def step(i, carry):
            off = base + i * _CHUNK
            pltpu.sync_copy(idx_hbm.at[pl.ds(off, _CHUNK)], idx_v)
            pltpu.async_copy(table_hbm.at[idx_v], rows_v, sem).wait()
            pltpu.sync_copy(rows_v, out_hbm.at[pl.ds(off, _CHUNK)])
            return carry

        lax.fori_loop(0, num_chunks, step, 0)

    return k(flat_idx, table)


def kernel(input_ids, table):
    Bt, H = input_ids.shape
    D = table.shape[1]
    flat_idx = input_ids.reshape(-1).astype(jnp.int32)
    B = Bt * H
    b_per_w = B // _NW
    num_chunks = b_per_w // _CHUNK
    out = _gather(flat_idx, table, b_per_w, num_chunks)
    return out.reshape(Bt, H, D)


# speedup vs baseline: 2.5662x; 1.0304x over previous
import functools

import jax
import jax.numpy as jnp
from jax import lax
from jax.experimental import pallas as pl
from jax.experimental.pallas import tpu as pltpu
from jax.experimental.pallas import tpu_sc as plsc

_INFO = plsc.get_sparse_core_info()
_NC = _INFO.num_cores
_NS = _INFO.num_subcores
_NW = _NC * _NS

_CHUNK = 1024
_NBUF = 4


@functools.partial(jax.jit, static_argnums=(2, 3))
def _gather(flat_idx, table, b_per_w, nchunks):
    D = table.shape[1]
    B = flat_idx.shape[0]
    ngroups = nchunks // _NBUF
    mesh = plsc.VectorSubcoreMesh(core_axis_name="c", subcore_axis_name="s")

    @functools.partial(
        pl.kernel,
        mesh=mesh,
        out_type=jax.ShapeDtypeStruct((B, D), jnp.float32),
        scratch_types=[
            pltpu.VMEM((_NBUF, _CHUNK), jnp.int32),
            pltpu.VMEM((_NBUF, _CHUNK, D), jnp.float32),
            pltpu.SemaphoreType.DMA((_NBUF,)),
            pltpu.SemaphoreType.DMA((_NBUF,)),
        ],
        compiler_params=pltpu.CompilerParams(use_tc_tiling_on_sc=False),
    )
    def k(idx_hbm, table_hbm, out_hbm, idx_v, rows_v, gsem, osem):
        wid = lax.axis_index("s") * _NC + lax.axis_index("c")
        base = wid * b_per_w

        def idx_in(g, b):
            off = base + (g * _NBUF + b) * _CHUNK
            pltpu.sync_copy(idx_hbm.at[pl.ds(off, _CHUNK)], idx_v.at[b])

        def gather(b):
            return pltpu.make_async_copy(
                table_hbm.at[idx_v.at[b]], rows_v.at[b], gsem.at[b])

        def out(g, b):
            off = base + (g * _NBUF + b) * _CHUNK
            return pltpu.make_async_copy(
                rows_v.at[b], out_hbm.at[pl.ds(off, _CHUNK)], osem.at[b])

        for b in range(_NBUF):
            idx_in(0, b)
            gather(b).start()
            if b >= 1:
                gather(b - 1).wait()
                out(0, b - 1).start()

        def group(g, carry):
            for b in range(_NBUF):
                out(g - 1, b).wait()
                idx_in(g, b)
                gather(b).start()
                bp = b - 1 if b >= 1 else _NBUF - 1
                gp = g if b >= 1 else g - 1
                gather(bp).wait()
                out(gp, bp).start()
            return carry

        lax.fori_loop(1, ngroups, group, 0)

        gather(_NBUF - 1).wait()
        out(ngroups - 1, _NBUF - 1).start()
        for b in range(_NBUF):
            out(ngroups - 1, b).wait()

    return k(flat_idx, table)


def kernel(input_ids, table):
    Bt, H = input_ids.shape
    D = table.shape[1]
    flat_idx = input_ids.reshape(-1).astype(jnp.int32)
    B = Bt * H
    b_per_w = B // _NW
    nchunks = b_per_w // _CHUNK
    out = _gather(flat_idx, table, b_per_w, nchunks)
    return out.reshape(Bt, H, D)
